# Initial kernel scaffold; baseline (speedup 1.0000x reference)
import jax
import jax.numpy as jnp
from jax.experimental import pallas as pl


def kernel(inputs, table):
    raise NotImplementedError("write your pallas kernel here")


# speedup vs baseline: 2.4233x; 2.4233x over previous
import functools
import math

import jax
import jax.numpy as jnp
from jax import lax
from jax.experimental import pallas as pl
from jax.experimental.pallas import tpu as pltpu
from jax.experimental.pallas import tpu_sc as plsc

_NC, _NS, _LANES = 2, 16, 16
_NW = _NC * _NS


def _build(n_rows: int, d: int, chunk: int):
    n_per_w = n_rows // _NW
    n_chunks = n_per_w // chunk
    scale = math.sqrt(d)
    mesh = plsc.VectorSubcoreMesh(core_axis_name="c", subcore_axis_name="s")

    @functools.partial(
        pl.kernel,
        mesh=mesh,
        out_type=jax.ShapeDtypeStruct((n_rows, d), jnp.float32),
        scratch_types=[
            pltpu.VMEM((n_per_w,), jnp.int32),
            pltpu.VMEM((chunk, d), jnp.float32),
            pltpu.SemaphoreType.DMA,
        ],
    )
    def gather_scale(idx_hbm, table_hbm, out_hbm, idx_v, rows_v, sem):
        wid = lax.axis_index("s") * _NC + lax.axis_index("c")
        base = wid * n_per_w
        pltpu.sync_copy(idx_hbm.at[pl.ds(base, n_per_w)], idx_v)

        @pl.loop(0, n_chunks)
        def _chunk(ci):
            off = ci * chunk
            pltpu.async_copy(
                table_hbm.at[idx_v.at[pl.ds(off, chunk)]], rows_v, sem
            ).wait()

            @pl.loop(0, chunk)
            def _row(r):
                for j in range(d // _LANES):
                    sl = pl.ds(j * _LANES, _LANES)
                    rows_v[r, sl] = rows_v[r, sl] * scale

            pltpu.sync_copy(rows_v, out_hbm.at[pl.ds(base + off, chunk)])

    return gather_scale


def kernel(inputs, table):
    b, l = inputs.shape
    v, d = table.shape
    idx = jnp.reshape(inputs, (b * l,)).astype(jnp.int32)
    fn = _build(b * l, d, chunk=128)
    out = fn(idx, table)
    return jnp.reshape(out, (b, l, d))


# speedup vs baseline: 2.8051x; 1.1575x over previous
import functools
import math

import jax
import jax.numpy as jnp
from jax import lax
from jax.experimental import pallas as pl
from jax.experimental.pallas import tpu as pltpu
from jax.experimental.pallas import tpu_sc as plsc

_NC, _NS, _LANES = 2, 16, 16
_NW = _NC * _NS
_NBUF = 4
_LOOKAHEAD = 2


def _build(n_rows: int, d: int, chunk: int):
    n_per_w = n_rows // _NW
    n_chunks = n_per_w // chunk
    assert n_chunks % _NBUF == 0
    scale = math.sqrt(d)
    mesh = plsc.VectorSubcoreMesh(core_axis_name="c", subcore_axis_name="s")

    @functools.partial(
        pl.kernel,
        mesh=mesh,
        out_type=jax.ShapeDtypeStruct((n_rows, d), jnp.float32),
        scratch_types=[
            pltpu.VMEM((n_per_w,), jnp.int32),
            pltpu.VMEM((_NBUF * chunk, d), jnp.float32),
            [pltpu.SemaphoreType.DMA] * _NBUF,
            [pltpu.SemaphoreType.DMA] * _NBUF,
        ],
    )
    def gather_scale(idx_hbm, table_hbm, out_hbm, idx_v, bufs, gsem, osem):
        wid = lax.axis_index("s") * _NC + lax.axis_index("c")
        base = wid * n_per_w
        pltpu.sync_copy(idx_hbm.at[pl.ds(base, n_per_w)], idx_v)

        def gather_desc(cid, b):
            return pltpu.make_async_copy(
                table_hbm.at[idx_v.at[pl.ds(cid * chunk, chunk)]],
                bufs.at[pl.ds(b * chunk, chunk)],
                gsem[b],
            )

        def out_desc(cid, b):
            return pltpu.make_async_copy(
                bufs.at[pl.ds(b * chunk, chunk)],
                out_hbm.at[pl.ds(base + cid * chunk, chunk)],
                osem[b],
            )

        for c in range(_LOOKAHEAD):
            gather_desc(c, c % _NBUF).start()

        @pl.loop(0, n_chunks, step=_NBUF)
        def _ring(ci):
            for b in range(_NBUF):
                cid = ci + b
                gather_desc(cid, b).wait()

                @pl.loop(0, chunk)
                def _row(r):
                    row = b * chunk + r
                    for j in range(d // _LANES):
                        sl = pl.ds(j * _LANES, _LANES)
                        bufs[row, sl] = bufs[row, sl] * scale

                out_desc(cid, b).start()

                nb = (b + _LOOKAHEAD) % _NBUF
                ncid = cid + _LOOKAHEAD

                @pl.when(jnp.logical_and(ncid >= _NBUF, ncid < n_chunks))
                def _():
                    out_desc(ncid - _NBUF, nb).wait()

                @pl.when(ncid < n_chunks)
                def _():
                    gather_desc(ncid, nb).start()

        for b in range(_NBUF):
            out_desc(n_chunks - _NBUF + b, b).wait()

    return gather_scale


def kernel(inputs, table):
    b, l = inputs.shape
    v, d = table.shape
    idx = jnp.reshape(inputs, (b * l,)).astype(jnp.int32)
    fn = _build(b * l, d, chunk=64)
    out = fn(idx, table)
    return jnp.reshape(out, (b, l, d))


# speedup vs baseline: 5.2080x; 1.8566x over previous
import functools
import math

import jax
import jax.numpy as jnp
from jax import lax
from jax.experimental import pallas as pl
from jax.experimental.pallas import tpu as pltpu
from jax.experimental.pallas import tpu_sc as plsc

_NC, _NS, _LANES = 2, 16, 16
_NW = _NC * _NS
_NBUF = 4
_LOOKAHEAD = 2
_BPC = 4


def _build(nb: int, l: int, d: int):
    n_rows = nb * l
    n_per_w = n_rows // _NW
    b_per_w = nb // _NW
    chunk = _BPC * l
    n_chunks = b_per_w // _BPC
    assert n_chunks % _NBUF == 0 and chunk % 8 == 0
    scale = math.sqrt(d)
    mesh = plsc.VectorSubcoreMesh(core_axis_name="c", subcore_axis_name="s")

    @functools.partial(
        pl.kernel,
        mesh=mesh,
        out_type=jax.ShapeDtypeStruct((nb, l, d), jnp.float32),
        scratch_types=[
            pltpu.VMEM((n_per_w,), jnp.int32),
            pltpu.VMEM((_NBUF * chunk, d), jnp.float32),
            [pltpu.SemaphoreType.DMA] * _NBUF,
            [pltpu.SemaphoreType.DMA] * _NBUF,
        ],
    )
    def gather_scale(idx_hbm, table_hbm, out_hbm, idx_v, bufs, gsem, osem):
        wid = lax.axis_index("s") * _NC + lax.axis_index("c")
        base = wid * n_per_w
        b_base = wid * b_per_w
        pltpu.sync_copy(idx_hbm.at[pl.ds(base, n_per_w)], idx_v)

        def gather_descs(cid, b):
            off = cid * chunk
            buf0 = b * chunk
            return [
                pltpu.make_async_copy(
                    table_hbm.at[idx_v.at[pl.ds(off, 128)]],
                    bufs.at[pl.ds(buf0, 128)],
                    gsem[b],
                ),
                pltpu.make_async_copy(
                    table_hbm.at[idx_v.at[pl.ds(off + 128, chunk - 128)]],
                    bufs.at[pl.ds(buf0 + 128, chunk - 128)],
                    gsem[b],
                ),
            ]

        def out_descs(cid, b):
            b0 = b_base + cid * _BPC
            return [
                pltpu.make_async_copy(
                    bufs.at[pl.ds(b * chunk + bb * l, l)],
                    out_hbm.at[b0 + bb],
                    osem[b],
                )
                for bb in range(_BPC)
            ]

        for c in range(_LOOKAHEAD):
            for g in gather_descs(c, c % _NBUF):
                g.start()

        @pl.loop(0, n_chunks, step=_NBUF)
        def _ring(ci):
            for b in range(_NBUF):
                cid = ci + b
                for g in gather_descs(cid, b):
                    g.wait()

                @pl.loop(0, chunk)
                def _row(r):
                    row = b * chunk + r
                    for j in range(d // _LANES):
                        sl = pl.ds(j * _LANES, _LANES)
                        bufs[row, sl] = bufs[row, sl] * scale

                for o in out_descs(cid, b):
                    o.start()

                nb_ = (b + _LOOKAHEAD) % _NBUF
                ncid = cid + _LOOKAHEAD

                @pl.when(jnp.logical_and(ncid >= _NBUF, ncid < n_chunks))
                def _():
                    for o in out_descs(ncid - _NBUF, nb_):
                        o.wait()

                @pl.when(ncid < n_chunks)
                def _():
                    for g in gather_descs(ncid, nb_):
                        g.start()

        for b in range(_NBUF):
            for o in out_descs(n_chunks - _NBUF + b, b):
                o.wait()

    return gather_scale


def kernel(inputs, table):
    nb, l = inputs.shape
    v, d = table.shape
    idx = jnp.reshape(inputs, (nb * l,)).astype(jnp.int32)
    fn = _build(nb, l, d)
    return fn(idx, table)


# speedup vs baseline: 9.1732x; 1.7614x over previous
import functools
import math

import jax
import jax.numpy as jnp
from jax import lax
from jax.experimental import pallas as pl
from jax.experimental.pallas import tpu as pltpu
from jax.experimental.pallas import tpu_sc as plsc

_NC, _NS, _LANES = 2, 16, 16
_NW = _NC * _NS
_NBUF = 4
_LOOKAHEAD = 2
_CHUNK = 160


def _build(n_rows: int, d: int):
    n_per_w = n_rows // _NW
    chunk = _CHUNK
    n_chunks = n_per_w // chunk
    assert n_chunks % _NBUF == 0 and n_per_w % chunk == 0
    scale = math.sqrt(d)
    mesh = plsc.VectorSubcoreMesh(core_axis_name="c", subcore_axis_name="s")

    @functools.partial(
        pl.kernel,
        mesh=mesh,
        out_type=jax.ShapeDtypeStruct((n_rows, d), jnp.float32),
        scratch_types=[
            pltpu.VMEM((n_per_w,), jnp.int32),
            pltpu.VMEM((_NBUF * chunk, d), jnp.float32),
            [pltpu.SemaphoreType.DMA] * _NBUF,
            [pltpu.SemaphoreType.DMA] * _NBUF,
        ],
    )
    def gather_scale(idx_hbm, table_hbm, out_hbm, idx_v, bufs, gsem, osem):
        wid = lax.axis_index("s") * _NC + lax.axis_index("c")
        base = wid * n_per_w
        pltpu.sync_copy(idx_hbm.at[pl.ds(base, n_per_w)], idx_v)

        def gather_descs(cid, b):
            off = cid * chunk
            buf0 = b * chunk
            return [
                pltpu.make_async_copy(
                    table_hbm.at[idx_v.at[pl.ds(off, 128)]],
                    bufs.at[pl.ds(buf0, 128)],
                    gsem[b],
                ),
                pltpu.make_async_copy(
                    table_hbm.at[idx_v.at[pl.ds(off + 128, chunk - 128)]],
                    bufs.at[pl.ds(buf0 + 128, chunk - 128)],
                    gsem[b],
                ),
            ]

        def out_desc(cid, b):
            return pltpu.make_async_copy(
                bufs.at[pl.ds(b * chunk, chunk)],
                out_hbm.at[pl.ds(base + cid * chunk, chunk)],
                osem[b],
            )

        for c in range(_LOOKAHEAD):
            for g in gather_descs(c, c % _NBUF):
                g.start()

        @pl.loop(0, n_chunks, step=_NBUF)
        def _ring(ci):
            for b in range(_NBUF):
                cid = ci + b
                for g in gather_descs(cid, b):
                    g.wait()

                @pl.loop(0, chunk)
                def _row(r):
                    row = b * chunk + r
                    for j in range(d // _LANES):
                        sl = pl.ds(j * _LANES, _LANES)
                        bufs[row, sl] = bufs[row, sl] * scale

                out_desc(cid, b).start()

                nb_ = (b + _LOOKAHEAD) % _NBUF
                ncid = cid + _LOOKAHEAD

                @pl.when(jnp.logical_and(ncid >= _NBUF, ncid < n_chunks))
                def _():
                    out_desc(ncid - _NBUF, nb_).wait()

                @pl.when(ncid < n_chunks)
                def _():
                    for g in gather_descs(ncid, nb_):
                        g.start()

        for b in range(_NBUF):
            out_desc(n_chunks - _NBUF + b, b).wait()

    return gather_scale


def kernel(inputs, table):
    nb, l = inputs.shape
    v, d = table.shape
    idx = jnp.reshape(jnp.transpose(inputs), (l * nb,)).astype(jnp.int32)
    fn = _build(l * nb, d)
    out = fn(idx, table)
    return jnp.transpose(jnp.reshape(out, (l, nb, d)), (1, 0, 2))
